# baseline (speedup 1.0000x reference)
import dataclasses

import jax
import jax.numpy as jnp
from jax import lax
from jax.experimental import pallas as pl
from jax.experimental.pallas import tpu as pltpu
from jax.experimental.pallas import tpu_sc as plsc

B = 8
N = 4096
HALF = 2048
NUM_TO_SAMPLE = 2048
EXCHANGE = 204
CUT = NUM_TO_SAMPLE - EXCHANGE
TAIL0 = 1840
TAILN = HALF - TAIL0
ONE_BITS = 0x3F800000
L = 16
NVEC = HALF // L
RADIX_BITS = 5
RADIX = 1 << RADIX_BITS
NPASS = 6


def _sc_body(cn_hbm, fps_hbm, lin_hbm, out_hbm,
             cn_v, fps_v, lin_v, k0, i0, k1, i1, pv, hist,
             tk, ti, outb, shared):
    c = lax.axis_index("c")
    s = lax.axis_index("s")
    item = c * 8 + s
    r = item >> 1
    h = item & 1
    active = s < 8
    lane = lax.broadcasted_iota(jnp.int32, (L,), 0)

    @pl.when(active)
    def _load_and_sort():
        pltpu.sync_copy(cn_hbm.at[pl.ds(r * N, N)], cn_v)
        pltpu.sync_copy(fps_hbm.at[pl.ds(r * N + h * HALF, HALF)], fps_v)
        pltpu.sync_copy(lin_hbm, lin_v)

        is_a = (jnp.zeros((L,), jnp.int32) + h) == 0

        @pl.loop(0, NVEC)
        def _build(m):
            jidx = lane * NVEC + m
            f = plsc.load_gather(fps_v, [jidx])
            sc = plsc.load_gather(lin_v, [f])
            cv = plsc.load_gather(cn_v, [jidx + h * HALF])
            bits = plsc.bitcast(sc * cv, jnp.int32)
            key = jnp.where(is_a, ONE_BITS - bits, bits)
            plsc.store_scatter(k0, [jidx], key)
            plsc.store_scatter(i0, [jidx], jidx)

        ones = jnp.ones((L,), jnp.int32)
        for p in range(NPASS):
            kin, iin, kout, iout = (k0, i0, k1, i1) if p % 2 == 0 else (k1, i1, k0, i0)
            shift = jnp.full((L,), p * RADIX_BITS, jnp.int32)
            mask5 = jnp.full((L,), RADIX - 1, jnp.int32)

            @pl.loop(0, RADIX)
            def _zero(z):
                hist[pl.ds(z * L, L)] = jnp.zeros((L,), jnp.int32)

            @pl.loop(0, NVEC)
            def _sweep1(m):
                cix = lane * NVEC + m
                k = plsc.load_gather(kin, [cix])
                d = lax.shift_right_logical(k, shift) & mask5
                hix = d * L + lane
                pv[pl.ds(m * L, L)] = plsc.load_gather(hist, [hix])
                plsc.addupdate_scatter(hist, [hix], ones)

            def _scan(z, base):
                row = hist[pl.ds(z * L, L)]
                cs = plsc.cumsum(row)
                hist[pl.ds(z * L, L)] = (cs - row) + base
                return base + jnp.sum(row)

            lax.fori_loop(0, RADIX, _scan, jnp.int32(0))

            @pl.loop(0, NVEC)
            def _sweep2(m):
                cix = lane * NVEC + m
                k = plsc.load_gather(kin, [cix])
                iv = plsc.load_gather(iin, [cix])
                d = lax.shift_right_logical(k, shift) & mask5
                pos = plsc.load_gather(hist, [d * L + lane]) + pv[pl.ds(m * L, L)]
                plsc.store_scatter(kout, [pos], k)
                plsc.store_scatter(iout, [pos], iv)

        @pl.when(h == 1)
        def _publish():
            pltpu.sync_copy(k0.at[pl.ds(TAIL0, TAILN)],
                            shared.at[pl.ds(r * 1024, TAILN)])
            pltpu.sync_copy(i0.at[pl.ds(TAIL0, TAILN)],
                            shared.at[pl.ds(r * 1024 + 256, TAILN)])

    plsc.subcore_barrier()

    @pl.when(active & (h == 0))
    def _merge_and_store():
        pltpu.sync_copy(shared.at[pl.ds(r * 1024, TAILN)], tk)
        pltpu.sync_copy(shared.at[pl.ds(r * 1024 + 256, TAILN)], ti)

        one_bits = jnp.full((L,), ONE_BITS, jnp.int32)

        @pl.loop(0, TAILN // L)
        def _tail(t):
            off = TAIL0 + t * L
            ka = k0[pl.ds(off, L)]
            ia = i0[pl.ds(off, L)]
            kb = tk[pl.ds(t * L, L)]
            ib = ti[pl.ds(t * L, L)]
            keep_b = ((lane + off) >= CUT) & (kb > (one_bits - ka))
            outb[pl.ds(t * L, L)] = jnp.where(keep_b, ib + HALF, ia)

        pltpu.sync_copy(i0.at[pl.ds(0, TAIL0)], out_hbm.at[pl.ds(r * HALF, TAIL0)])
        pltpu.sync_copy(outb, out_hbm.at[pl.ds(r * HALF + TAIL0, TAILN)])


@jax.jit
def kernel(x, curv, fps_idxs):
    del x
    curv_n = curv - curv.min(axis=1)[..., None]
    curv_n = curv_n / curv_n.max(axis=1)[..., None]
    lin = jnp.linspace(1.0, 0.0, N, dtype=jnp.float32)

    mesh = plsc.VectorSubcoreMesh(core_axis_name="c", subcore_axis_name="s")
    cp = pltpu.CompilerParams()
    if "needs_layout_passes" in pltpu.CompilerParams.__dataclass_fields__:
        cp = dataclasses.replace(cp, needs_layout_passes=False)
    run = pl.kernel(
        _sc_body,
        compiler_params=cp,
        out_type=jax.ShapeDtypeStruct((B * HALF,), jnp.int32),
        mesh=mesh,
        scratch_types=[
            pltpu.VMEM((N,), jnp.float32),
            pltpu.VMEM((HALF,), jnp.int32),
            pltpu.VMEM((N,), jnp.float32),
            pltpu.VMEM((HALF,), jnp.int32),
            pltpu.VMEM((HALF,), jnp.int32),
            pltpu.VMEM((HALF,), jnp.int32),
            pltpu.VMEM((HALF,), jnp.int32),
            pltpu.VMEM((HALF,), jnp.int32),
            pltpu.VMEM((RADIX * L,), jnp.int32),
            pltpu.VMEM((TAILN,), jnp.int32),
            pltpu.VMEM((TAILN,), jnp.int32),
            pltpu.VMEM((TAILN,), jnp.int32),
            pltpu.VMEM_SHARED((B * 1024,), jnp.int32),
        ],
    )
    out = run(curv_n.reshape(B * N), fps_idxs.astype(jnp.int32).reshape(B * N), lin)
    return out.reshape(B, NUM_TO_SAMPLE)


# speedup vs baseline: 1.0828x; 1.0828x over previous
import dataclasses

import jax
import jax.numpy as jnp
from jax import lax
from jax.experimental import pallas as pl
from jax.experimental.pallas import tpu as pltpu
from jax.experimental.pallas import tpu_sc as plsc

B = 8
N = 4096
HALF = 2048
NUM_TO_SAMPLE = 2048
EXCHANGE = 204
CUT = NUM_TO_SAMPLE - EXCHANGE
TAIL0 = 1840
TAILN = HALF - TAIL0
ONE_BITS = 0x3F800000
L = 16
NVEC = HALF // L
RADIX_BITS = 5
RADIX = 1 << RADIX_BITS
NPASS = 6


def _sc_body(cn_hbm, fps_hbm, lin_hbm, out_hbm,
             cn_v, fps_v, lin_v, k0, i0, k1, i1, hist,
             tk, ti, outb, shared):
    c = lax.axis_index("c")
    s = lax.axis_index("s")
    item = c * 8 + s
    r = item >> 1
    h = item & 1
    active = s < 8
    lane = lax.broadcasted_iota(jnp.int32, (L,), 0)

    @pl.when(active)
    def _load_and_sort():
        pltpu.sync_copy(cn_hbm.at[pl.ds(r * N, N)], cn_v)
        pltpu.sync_copy(fps_hbm.at[pl.ds(r * N + h * HALF, HALF)], fps_v)
        pltpu.sync_copy(lin_hbm, lin_v)

        is_a = (jnp.zeros((L,), jnp.int32) + h) == 0

        @pl.loop(0, NVEC, unroll=4)
        def _build(m):
            jidx = lane * NVEC + m
            f = plsc.load_gather(fps_v, [jidx])
            sc = plsc.load_gather(lin_v, [f])
            cv = plsc.load_gather(cn_v, [jidx + h * HALF])
            bits = plsc.bitcast(sc * cv, jnp.int32)
            key = jnp.where(is_a, ONE_BITS - bits, bits)
            plsc.store_scatter(k0, [jidx], key)

        ones = jnp.ones((L,), jnp.int32)
        for p in range(NPASS):
            kin, iin, kout, iout = (k0, i0, k1, i1) if p % 2 == 0 else (k1, i1, k0, i0)
            shift = jnp.full((L,), p * RADIX_BITS, jnp.int32)
            mask5 = jnp.full((L,), RADIX - 1, jnp.int32)

            @pl.loop(0, RADIX, unroll=4)
            def _zero(z):
                hist[pl.ds(z * L, L)] = jnp.zeros((L,), jnp.int32)

            @pl.loop(0, NVEC, unroll=4)
            def _sweep1(m):
                cix = lane * NVEC + m
                k = plsc.load_gather(kin, [cix])
                d = lax.shift_right_logical(k, shift) & mask5
                plsc.addupdate_scatter(hist, [d * L + lane], ones)

            def _scan(z, base):
                row = hist[pl.ds(z * L, L)]
                cs = plsc.cumsum(row)
                hist[pl.ds(z * L, L)] = (cs - row) + base
                return base + jnp.sum(row)

            lax.fori_loop(0, RADIX, _scan, jnp.int32(0))

            @pl.loop(0, NVEC, unroll=2)
            def _sweep2(m):
                cix = lane * NVEC + m
                k = plsc.load_gather(kin, [cix])
                iv = cix if p == 0 else plsc.load_gather(iin, [cix])
                d = lax.shift_right_logical(k, shift) & mask5
                hix = d * L + lane
                pos = plsc.load_gather(hist, [hix])
                plsc.store_scatter(kout, [pos], k)
                plsc.store_scatter(iout, [pos], iv)
                plsc.addupdate_scatter(hist, [hix], ones)

        @pl.when(h == 1)
        def _publish():
            pltpu.sync_copy(k0.at[pl.ds(TAIL0, TAILN)],
                            shared.at[pl.ds(r * 1024, TAILN)])
            pltpu.sync_copy(i0.at[pl.ds(TAIL0, TAILN)],
                            shared.at[pl.ds(r * 1024 + 256, TAILN)])

    plsc.subcore_barrier()

    @pl.when(active & (h == 0))
    def _merge_and_store():
        pltpu.sync_copy(shared.at[pl.ds(r * 1024, TAILN)], tk)
        pltpu.sync_copy(shared.at[pl.ds(r * 1024 + 256, TAILN)], ti)

        one_bits = jnp.full((L,), ONE_BITS, jnp.int32)

        @pl.loop(0, TAILN // L)
        def _tail(t):
            off = TAIL0 + t * L
            ka = k0[pl.ds(off, L)]
            ia = i0[pl.ds(off, L)]
            kb = tk[pl.ds(t * L, L)]
            ib = ti[pl.ds(t * L, L)]
            keep_b = ((lane + off) >= CUT) & (kb > (one_bits - ka))
            outb[pl.ds(t * L, L)] = jnp.where(keep_b, ib + HALF, ia)

        pltpu.sync_copy(i0.at[pl.ds(0, TAIL0)], out_hbm.at[pl.ds(r * HALF, TAIL0)])
        pltpu.sync_copy(outb, out_hbm.at[pl.ds(r * HALF + TAIL0, TAILN)])


@jax.jit
def kernel(x, curv, fps_idxs):
    del x
    curv_n = curv - curv.min(axis=1)[..., None]
    curv_n = curv_n / curv_n.max(axis=1)[..., None]
    lin = jnp.linspace(1.0, 0.0, N, dtype=jnp.float32)

    mesh = plsc.VectorSubcoreMesh(core_axis_name="c", subcore_axis_name="s")
    cp = pltpu.CompilerParams()
    if "needs_layout_passes" in pltpu.CompilerParams.__dataclass_fields__:
        cp = dataclasses.replace(cp, needs_layout_passes=False)
    run = pl.kernel(
        _sc_body,
        compiler_params=cp,
        out_type=jax.ShapeDtypeStruct((B * HALF,), jnp.int32),
        mesh=mesh,
        scratch_types=[
            pltpu.VMEM((N,), jnp.float32),
            pltpu.VMEM((HALF,), jnp.int32),
            pltpu.VMEM((N,), jnp.float32),
            pltpu.VMEM((HALF,), jnp.int32),
            pltpu.VMEM((HALF,), jnp.int32),
            pltpu.VMEM((HALF,), jnp.int32),
            pltpu.VMEM((HALF,), jnp.int32),
            pltpu.VMEM((RADIX * L,), jnp.int32),
            pltpu.VMEM((TAILN,), jnp.int32),
            pltpu.VMEM((TAILN,), jnp.int32),
            pltpu.VMEM((TAILN,), jnp.int32),
            pltpu.VMEM_SHARED((B * 1024,), jnp.int32),
        ],
    )
    out = run(curv_n.reshape(B * N), fps_idxs.astype(jnp.int32).reshape(B * N), lin)
    return out.reshape(B, NUM_TO_SAMPLE)


# speedup vs baseline: 1.6795x; 1.5511x over previous
import dataclasses

import jax
import jax.numpy as jnp
from jax import lax
from jax.experimental import pallas as pl
from jax.experimental.pallas import tpu as pltpu
from jax.experimental.pallas import tpu_sc as plsc

B = 8
N = 4096
HALF = 2048
NUM_TO_SAMPLE = 2048
EXCHANGE = 204
CUT = NUM_TO_SAMPLE - EXCHANGE
TAIL0 = 1840
TAILN = HALF - TAIL0
ONE_BITS = 0x3F800000
L = 16
NVEC = HALF // L
RADIX_BITS = 8
RADIX = 1 << RADIX_BITS
NPASS = 4
CNT_ONE_BASED = True


def _sc_body(cn_hbm, fps_hbm, lin_hbm, out_hbm,
             cn_v, fps_v, lin_v, k0, i0, k1, i1, hist,
             tk, ti, outb, shared):
    c = lax.axis_index("c")
    s = lax.axis_index("s")
    item = c * 8 + s
    r = item >> 1
    h = item & 1
    active = s < 8
    lane = lax.broadcasted_iota(jnp.int32, (L,), 0)

    @pl.when(active)
    def _load_and_sort():
        pltpu.sync_copy(cn_hbm.at[pl.ds(r * N, N)], cn_v)
        pltpu.sync_copy(fps_hbm.at[pl.ds(r * N + h * HALF, HALF)], fps_v)
        pltpu.sync_copy(lin_hbm, lin_v)

        is_a = (jnp.zeros((L,), jnp.int32) + h) == 0

        @pl.loop(0, NVEC, unroll=4)
        def _build(m):
            f = fps_v[pl.ds(m * L, L)]
            sc = plsc.load_gather(lin_v, [f])
            cv = cn_v[pl.ds(h * HALF + m * L, L)]
            bits = plsc.bitcast(sc * cv, jnp.int32)
            key = jnp.where(is_a, ONE_BITS - bits, bits)
            k0[pl.ds(m * L, L)] = key

        for p in range(NPASS):
            kin, iin, kout, iout = (k0, i0, k1, i1) if p % 2 == 0 else (k1, i1, k0, i0)
            shift = jnp.full((L,), p * RADIX_BITS, jnp.int32)
            dmask = jnp.full((L,), RADIX - 1, jnp.int32)

            for z in range(RADIX // L):
                hist[pl.ds(z * L, L)] = jnp.zeros((L,), jnp.int32)

            @pl.loop(0, NVEC, unroll=4)
            def _sweep1(m):
                k = kin[pl.ds(m * L, L)]
                d = lax.shift_right_logical(k, shift) & dmask
                cnt, lastm = plsc.scan_count(d)
                tot = cnt if CNT_ONE_BASED else cnt + 1
                plsc.addupdate_scatter(hist, [d], tot, mask=lastm)

            base = jnp.int32(0)
            for z in range(RADIX // L):
                row = hist[pl.ds(z * L, L)]
                cs = plsc.cumsum(row)
                hist[pl.ds(z * L, L)] = (cs - row) + base
                base = base + jnp.sum(row)

            @pl.loop(0, NVEC, unroll=2)
            def _sweep2(m):
                k = kin[pl.ds(m * L, L)]
                iv = (lane + m * L) if p == 0 else iin[pl.ds(m * L, L)]
                d = lax.shift_right_logical(k, shift) & dmask
                cnt, lastm = plsc.scan_count(d)
                prior = (cnt - 1) if CNT_ONE_BASED else cnt
                pos = plsc.load_gather(hist, [d]) + prior
                plsc.store_scatter(kout, [pos], k)
                plsc.store_scatter(iout, [pos], iv)
                tot = cnt if CNT_ONE_BASED else cnt + 1
                plsc.addupdate_scatter(hist, [d], tot, mask=lastm)

        @pl.when(h == 1)
        def _publish():
            pltpu.sync_copy(k0.at[pl.ds(TAIL0, TAILN)],
                            shared.at[pl.ds(r * 1024, TAILN)])
            pltpu.sync_copy(i0.at[pl.ds(TAIL0, TAILN)],
                            shared.at[pl.ds(r * 1024 + 256, TAILN)])

    plsc.subcore_barrier()

    @pl.when(active & (h == 0))
    def _merge_and_store():
        pltpu.sync_copy(shared.at[pl.ds(r * 1024, TAILN)], tk)
        pltpu.sync_copy(shared.at[pl.ds(r * 1024 + 256, TAILN)], ti)

        one_bits = jnp.full((L,), ONE_BITS, jnp.int32)

        @pl.loop(0, TAILN // L)
        def _tail(t):
            off = TAIL0 + t * L
            ka = k0[pl.ds(off, L)]
            ia = i0[pl.ds(off, L)]
            kb = tk[pl.ds(t * L, L)]
            ib = ti[pl.ds(t * L, L)]
            keep_b = ((lane + off) >= CUT) & (kb > (one_bits - ka))
            outb[pl.ds(t * L, L)] = jnp.where(keep_b, ib + HALF, ia)

        pltpu.sync_copy(i0.at[pl.ds(0, TAIL0)], out_hbm.at[pl.ds(r * HALF, TAIL0)])
        pltpu.sync_copy(outb, out_hbm.at[pl.ds(r * HALF + TAIL0, TAILN)])


@jax.jit
def kernel(x, curv, fps_idxs):
    del x
    curv_n = curv - curv.min(axis=1)[..., None]
    curv_n = curv_n / curv_n.max(axis=1)[..., None]
    lin = jnp.linspace(1.0, 0.0, N, dtype=jnp.float32)

    mesh = plsc.VectorSubcoreMesh(core_axis_name="c", subcore_axis_name="s")
    cp = pltpu.CompilerParams()
    if "needs_layout_passes" in pltpu.CompilerParams.__dataclass_fields__:
        cp = dataclasses.replace(cp, needs_layout_passes=False)
    run = pl.kernel(
        _sc_body,
        compiler_params=cp,
        out_type=jax.ShapeDtypeStruct((B * HALF,), jnp.int32),
        mesh=mesh,
        scratch_types=[
            pltpu.VMEM((N,), jnp.float32),
            pltpu.VMEM((HALF,), jnp.int32),
            pltpu.VMEM((N,), jnp.float32),
            pltpu.VMEM((HALF,), jnp.int32),
            pltpu.VMEM((HALF,), jnp.int32),
            pltpu.VMEM((HALF,), jnp.int32),
            pltpu.VMEM((HALF,), jnp.int32),
            pltpu.VMEM((RADIX,), jnp.int32),
            pltpu.VMEM((TAILN,), jnp.int32),
            pltpu.VMEM((TAILN,), jnp.int32),
            pltpu.VMEM((TAILN,), jnp.int32),
            pltpu.VMEM_SHARED((B * 1024,), jnp.int32),
        ],
    )
    out = run(curv_n.reshape(B * N), fps_idxs.astype(jnp.int32).reshape(B * N), lin)
    return out.reshape(B, NUM_TO_SAMPLE)


# speedup vs baseline: 1.7450x; 1.0390x over previous
import dataclasses

import jax
import jax.numpy as jnp
from jax import lax
from jax.experimental import pallas as pl
from jax.experimental.pallas import tpu as pltpu
from jax.experimental.pallas import tpu_sc as plsc

B = 8
N = 4096
HALF = 2048
NUM_TO_SAMPLE = 2048
EXCHANGE = 204
CUT = NUM_TO_SAMPLE - EXCHANGE
TAIL0 = 1840
TAILN = HALF - TAIL0
ONE_BITS = 0x3F800000
L = 16
NVEC = HALF // L
RADIX_BITS = 8
RADIX = 1 << RADIX_BITS
NPASS = 4
CNT_ONE_BASED = True


def _sc_body(cn_hbm, fps_hbm, lin_hbm, out_hbm,
             cn_v, fps_v, lin_v, k0, i0, k1, i1, hist,
             tk, ti, outb, shared):
    s = lax.axis_index("s")
    item = s
    r = item >> 1
    h = item & 1
    active = s < 16
    lane = lax.broadcasted_iota(jnp.int32, (L,), 0)

    @pl.when(active)
    def _load_and_sort():
        pltpu.sync_copy(cn_hbm.at[pl.ds(r * N, N)], cn_v)
        pltpu.sync_copy(fps_hbm.at[pl.ds(r * N + h * HALF, HALF)], fps_v)
        pltpu.sync_copy(lin_hbm, lin_v)

        is_a = (jnp.zeros((L,), jnp.int32) + h) == 0

        @pl.loop(0, NVEC, unroll=4)
        def _build(m):
            f = fps_v[pl.ds(m * L, L)]
            sc = plsc.load_gather(lin_v, [f])
            cv = cn_v[pl.ds(h * HALF + m * L, L)]
            bits = plsc.bitcast(sc * cv, jnp.int32)
            key = jnp.where(is_a, ONE_BITS - bits, bits)
            k0[pl.ds(m * L, L)] = key

        for p in range(NPASS):
            kin, iin, kout, iout = (k0, i0, k1, i1) if p % 2 == 0 else (k1, i1, k0, i0)
            shift = jnp.full((L,), p * RADIX_BITS, jnp.int32)
            dmask = jnp.full((L,), RADIX - 1, jnp.int32)

            for z in range(RADIX // L):
                hist[pl.ds(z * L, L)] = jnp.zeros((L,), jnp.int32)

            @pl.loop(0, NVEC, unroll=4)
            def _sweep1(m):
                k = kin[pl.ds(m * L, L)]
                d = lax.shift_right_logical(k, shift) & dmask
                cnt, lastm = plsc.scan_count(d)
                tot = cnt if CNT_ONE_BASED else cnt + 1
                plsc.addupdate_scatter(hist, [d], tot, mask=lastm)

            base = jnp.int32(0)
            for z in range(RADIX // L):
                row = hist[pl.ds(z * L, L)]
                cs = plsc.cumsum(row)
                hist[pl.ds(z * L, L)] = (cs - row) + base
                base = base + jnp.sum(row)

            @pl.loop(0, NVEC, unroll=2)
            def _sweep2(m):
                k = kin[pl.ds(m * L, L)]
                iv = (lane + m * L) if p == 0 else iin[pl.ds(m * L, L)]
                d = lax.shift_right_logical(k, shift) & dmask
                cnt, lastm = plsc.scan_count(d)
                prior = (cnt - 1) if CNT_ONE_BASED else cnt
                pos = plsc.load_gather(hist, [d]) + prior
                plsc.store_scatter(kout, [pos], k)
                plsc.store_scatter(iout, [pos], iv)
                tot = cnt if CNT_ONE_BASED else cnt + 1
                plsc.addupdate_scatter(hist, [d], tot, mask=lastm)

        @pl.when(h == 1)
        def _publish():
            pltpu.sync_copy(k0.at[pl.ds(TAIL0, TAILN)],
                            shared.at[pl.ds(r * 1024, TAILN)])
            pltpu.sync_copy(i0.at[pl.ds(TAIL0, TAILN)],
                            shared.at[pl.ds(r * 1024 + 256, TAILN)])

    plsc.subcore_barrier()

    @pl.when(active & (h == 0))
    def _merge_and_store():
        pltpu.sync_copy(shared.at[pl.ds(r * 1024, TAILN)], tk)
        pltpu.sync_copy(shared.at[pl.ds(r * 1024 + 256, TAILN)], ti)

        one_bits = jnp.full((L,), ONE_BITS, jnp.int32)

        @pl.loop(0, TAILN // L)
        def _tail(t):
            off = TAIL0 + t * L
            ka = k0[pl.ds(off, L)]
            ia = i0[pl.ds(off, L)]
            kb = tk[pl.ds(t * L, L)]
            ib = ti[pl.ds(t * L, L)]
            keep_b = ((lane + off) >= CUT) & (kb > (one_bits - ka))
            outb[pl.ds(t * L, L)] = jnp.where(keep_b, ib + HALF, ia)

        pltpu.sync_copy(i0.at[pl.ds(0, TAIL0)], out_hbm.at[pl.ds(r * HALF, TAIL0)])
        pltpu.sync_copy(outb, out_hbm.at[pl.ds(r * HALF + TAIL0, TAILN)])


@jax.jit
def kernel(x, curv, fps_idxs):
    del x
    curv_n = curv - curv.min(axis=1)[..., None]
    curv_n = curv_n / curv_n.max(axis=1)[..., None]
    lin = jnp.linspace(1.0, 0.0, N, dtype=jnp.float32)

    mesh = plsc.VectorSubcoreMesh(core_axis_name="c", subcore_axis_name="s",
                                  num_cores=1)
    cp = pltpu.CompilerParams()
    if "needs_layout_passes" in pltpu.CompilerParams.__dataclass_fields__:
        cp = dataclasses.replace(cp, needs_layout_passes=False)
    run = pl.kernel(
        _sc_body,
        compiler_params=cp,
        out_type=jax.ShapeDtypeStruct((B * HALF,), jnp.int32),
        mesh=mesh,
        scratch_types=[
            pltpu.VMEM((N,), jnp.float32),
            pltpu.VMEM((HALF,), jnp.int32),
            pltpu.VMEM((N,), jnp.float32),
            pltpu.VMEM((HALF,), jnp.int32),
            pltpu.VMEM((HALF,), jnp.int32),
            pltpu.VMEM((HALF,), jnp.int32),
            pltpu.VMEM((HALF,), jnp.int32),
            pltpu.VMEM((RADIX,), jnp.int32),
            pltpu.VMEM((TAILN,), jnp.int32),
            pltpu.VMEM((TAILN,), jnp.int32),
            pltpu.VMEM((TAILN,), jnp.int32),
            pltpu.VMEM_SHARED((B * 1024,), jnp.int32),
        ],
    )
    out = run(curv_n.reshape(B * N), fps_idxs.astype(jnp.int32).reshape(B * N), lin)
    return out.reshape(B, NUM_TO_SAMPLE)
